# baseline (speedup 1.0000x reference)
import functools

import jax
import jax.numpy as jnp
from jax import lax
from jax.experimental import pallas as pl
from jax.experimental.pallas import tpu as pltpu
from jax.experimental.pallas import tpu_sc as plsc

_N = 10000
_NPAD = 10240
_ACC = 2 * _NPAD
_E = 320000
_EPAD = 327680
_CH = 128
_EPT = _EPAD // 16
_NCHUNK = _EPT // _CH
_BLKCH = 8
_NBLK = _NCHUNK // _BLKCH
_SLICE = _ACC // 16
_B = 4096

@functools.cache
def _sc_mesh():
    return plsc.VectorSubcoreMesh(core_axis_name="c", subcore_axis_name="s")


@functools.cache
def _make_agg(do_counts):
    out_type = [jax.ShapeDtypeStruct((2 * _ACC, 64), jnp.float32)]
    if do_counts:
        out_type.append(jax.ShapeDtypeStruct((2 * _ACC, 16), jnp.float32))

    blk_e = _BLKCH * _CH
    blk_r = blk_e // _CH
    nring = 2 if do_counts else 4
    scratch = [
        pltpu.VMEM((blk_e,), jnp.int32),
        pltpu.VMEM((blk_r, _CH), jnp.int32),
        pltpu.VMEM((blk_r, _CH), jnp.int32),
        pltpu.VMEM((nring, _CH, 64), jnp.float32),
        pltpu.VMEM_SHARED((_ACC, 64), jnp.float32),
        pltpu.SemaphoreType.DMA,
        pltpu.SemaphoreType.DMA,
    ]
    if do_counts:
        scratch += [
            pltpu.VMEM((_CH, 16), jnp.float32),
            pltpu.VMEM((_CH, 16), jnp.float32),
            pltpu.VMEM_SHARED((_ACC, 16), jnp.float32),
            pltpu.SemaphoreType.DMA,
        ]

    def body(table, srch, dsth, typh, *rest):
        if do_counts:
            (out_agg, out_cnt, srcb, dstb, typb, rows, acc, sem_g,
             sem_s, ones, z16, cntacc, sem_c) = rest
        else:
            out_agg, srcb, dstb, typb, rows, acc, sem_g, sem_s = rest

        c = lax.axis_index("c")
        s = lax.axis_index("s")

        zeros16 = jnp.zeros((16,), jnp.float32)
        ones16 = jnp.ones((16,), jnp.float32)

        def init_buf(i, carry):
            for j in range(4):
                rows[0, i, pl.ds(16 * j, 16)] = zeros16
            if do_counts:
                ones[i, pl.ds(0, 16)] = ones16
                z16[i, pl.ds(0, 16)] = zeros16
            return carry

        lax.fori_loop(0, _CH, init_buf, 0)

        base_e = s * _EPT
        base_r = s * (_EPT // _CH)

        def zero_acc(i, carry):
            base = s * _SLICE + i * _CH
            pltpu.sync_copy(rows.at[0], acc.at[pl.ds(base, _CH)])
            if do_counts:
                pltpu.sync_copy(z16, cntacc.at[pl.ds(base, _CH)])
            return carry

        lax.fori_loop(0, _SLICE // _CH, zero_acc, 0)
        plsc.subcore_barrier()

        halfblk = _NBLK // 2

        def block(blk, carry):
            e0 = base_e + blk * blk_e
            r0 = base_r + blk * blk_r
            pltpu.sync_copy(srch.at[pl.ds(e0, blk_e)], srcb)
            pltpu.sync_copy(dsth.at[pl.ds(r0, blk_r)], dstb)
            pltpu.sync_copy(typh.at[pl.ds(r0, blk_r)], typb)
            for v in range(blk_e // 16):
                sl = pl.ds(16 * v, 16)
                slm = pl.ds(16 * (v % 8), 16)
                srcb[sl] = srcb[sl] * 2 + c
                dstb[v // 8, slm] = typb[v // 8, slm] * _NPAD + dstb[v // 8, slm]
            combb = dstb
            if do_counts:
                counting = (((c == 0) & (blk < halfblk))
                            | ((c == 1) & (blk >= halfblk)))

                @pl.when(counting)
                def _cnt_fire():
                    for j in range(_BLKCH):
                        pltpu.async_copy(ones, cntacc.at[combb.at[j]],
                                         sem_c, add=True)
            gd = [None] * _BLKCH
            sd = [None] * _BLKCH
            for j in range(_BLKCH):
                if j >= nring:
                    sd[j - nring].wait()
                gd[j] = pltpu.async_copy(
                    table.at[srcb.at[pl.ds(j * _CH, _CH)]],
                    rows.at[j % nring], sem_g)
                if j >= 1:
                    gd[j - 1].wait()
                    sd[j - 1] = pltpu.async_copy(
                        rows.at[(j - 1) % nring], acc.at[combb.at[j - 1]],
                        sem_s, add=True)
            gd[_BLKCH - 1].wait()
            sd[_BLKCH - 1] = pltpu.async_copy(
                rows.at[(_BLKCH - 1) % nring], acc.at[combb.at[_BLKCH - 1]],
                sem_s, add=True)
            for j in range(_BLKCH - nring, _BLKCH):
                sd[j].wait()
            if do_counts:
                @pl.when(counting)
                def _cnt_drain():
                    for j in range(_BLKCH):
                        pltpu.make_async_copy(
                            ones, cntacc.at[combb.at[j]], sem_c).wait()
            return carry

        lax.fori_loop(0, _NBLK, block, 0)
        plsc.subcore_barrier()

        obase = c * _ACC + s * _SLICE
        pltpu.sync_copy(acc.at[pl.ds(s * _SLICE, _SLICE)],
                        out_agg.at[pl.ds(obase, _SLICE)])
        if do_counts:
            pltpu.sync_copy(cntacc.at[pl.ds(s * _SLICE, _SLICE)],
                            out_cnt.at[pl.ds(obase, _SLICE)])

    return pl.kernel(body, out_type, mesh=_sc_mesh(), scratch_types=scratch,
                     compiler_params=pltpu.CompilerParams(
                         use_tc_tiling_on_sc=False))


@functools.cache
def _make_expert_gather():
    rows_per_tile = _B // 32

    def body(table, idxh, out, idxb, rows, sem):
        c = lax.axis_index("c")
        s = lax.axis_index("s")
        wid = s * 2 + c
        base = wid * rows_per_tile
        pltpu.sync_copy(idxh.at[pl.ds(base, rows_per_tile)], idxb)
        pltpu.async_copy(table.at[idxb], rows, sem).wait()
        pltpu.sync_copy(rows, out.at[pl.ds(base, rows_per_tile)])

    return pl.kernel(
        body,
        jax.ShapeDtypeStruct((_B, 64), jnp.float32),
        mesh=_sc_mesh(),
        scratch_types=[
            pltpu.VMEM((rows_per_tile,), jnp.int32),
            pltpu.VMEM((rows_per_tile, 64), jnp.float32),
            pltpu.SemaphoreType.DMA,
        ],
        compiler_params=pltpu.CompilerParams(use_tc_tiling_on_sc=False),
    )


def _combine_body(x_ref, agg_ref, cnt_ref, root_ref, basis_ref, comp_ref,
                  b_ref, o_ref, *, relu, split_out, x_interleaved):
    if x_interleaved:
        x = jnp.concatenate([x_ref[:, 0, :], x_ref[:, 1, :]], axis=1)
    else:
        x = x_ref[...]
    acc = jnp.dot(x, root_ref[...], preferred_element_type=jnp.float32)
    acc = acc + b_ref[...]
    for r in range(2):
        w_r = (comp_ref[r:r + 1, 0:1] * basis_ref[0]
               + comp_ref[r:r + 1, 1:2] * basis_ref[1])
        cr = cnt_ref[0, r, :, 0:1] + cnt_ref[1, r, :, 0:1]
        denom = jnp.maximum(cr, 1.0)
        for h in range(2):
            mean = agg_ref[h, r] / denom
            acc = acc + jnp.dot(mean, w_r[64 * h:64 * (h + 1), :],
                                preferred_element_type=jnp.float32)
    if relu:
        acc = jnp.maximum(acc, 0.0)
    if split_out:
        o_ref[:, 0, :] = acc[:, :64]
        o_ref[:, 1, :] = acc[:, 64:]
    else:
        o_ref[...] = acc


def _make_combine(d_out, relu, split_out, x_interleaved, bn=1000):
    grid = (_N // bn,)
    if split_out:
        out_shape = jax.ShapeDtypeStruct((_N, 2, 64), jnp.float32)
        out_spec = pl.BlockSpec((bn, 2, 64), lambda i: (i, 0, 0))
    else:
        out_shape = jax.ShapeDtypeStruct((_N, d_out), jnp.float32)
        out_spec = pl.BlockSpec((bn, d_out), lambda i: (i, 0))
    if x_interleaved:
        x_spec = pl.BlockSpec((bn, 2, 64), lambda i: (i, 0, 0))
    else:
        x_spec = pl.BlockSpec((bn, 128), lambda i: (i, 0))
    return pl.pallas_call(
        functools.partial(_combine_body, relu=relu, split_out=split_out,
                          x_interleaved=x_interleaved),
        grid=grid,
        in_specs=[
            x_spec,
            pl.BlockSpec((2, 2, bn, 64), lambda i: (0, 0, i, 0)),
            pl.BlockSpec((2, 2, bn, 16), lambda i: (0, 0, i, 0)),
            pl.BlockSpec((128, d_out), lambda i: (0, 0)),
            pl.BlockSpec((2, 128, d_out), lambda i: (0, 0, 0)),
            pl.BlockSpec((2, 2), lambda i: (0, 0)),
            pl.BlockSpec((1, d_out), lambda i: (0, 0)),
        ],
        out_specs=out_spec,
        out_shape=out_shape,
    )


_combine1 = _make_combine(128, relu=True, split_out=True, x_interleaved=False)
_combine2 = _make_combine(64, relu=False, split_out=False, x_interleaved=True)


def _classifier_body(e_ref, w1_ref, b1_ref, g1_ref, bb1_ref,
                     w2_ref, b2_ref, g2_ref, bb2_ref, w3_ref, b3_ref, o_ref):
    z = jnp.dot(e_ref[...], w1_ref[...], preferred_element_type=jnp.float32)
    z = z + b1_ref[...]
    mu = jnp.mean(z, axis=1, keepdims=True)
    zc = z - mu
    var = jnp.mean(zc * zc, axis=1, keepdims=True)
    z = zc * lax.rsqrt(var + 1e-5) * g1_ref[...] + bb1_ref[...]
    z = jnp.maximum(z, 0.0)
    z = jnp.dot(z, w2_ref[...], preferred_element_type=jnp.float32)
    z = z + b2_ref[...]
    mu = jnp.mean(z, axis=1, keepdims=True)
    zc = z - mu
    var = jnp.mean(zc * zc, axis=1, keepdims=True)
    z = zc * lax.rsqrt(var + 1e-5) * g2_ref[...] + bb2_ref[...]
    z = jnp.maximum(z, 0.0)
    p = jnp.sum(z * w3_ref[...], axis=1, keepdims=True) + b3_ref[...]
    o_ref[...] = jax.nn.sigmoid(p)


def _make_classifier(bn=512):
    grid = (_B // bn,)
    return pl.pallas_call(
        _classifier_body,
        grid=grid,
        in_specs=[
            pl.BlockSpec((bn, 64), lambda i: (i, 0)),
            pl.BlockSpec((64, 64), lambda i: (0, 0)),
            pl.BlockSpec((1, 64), lambda i: (0, 0)),
            pl.BlockSpec((1, 64), lambda i: (0, 0)),
            pl.BlockSpec((1, 64), lambda i: (0, 0)),
            pl.BlockSpec((64, 32), lambda i: (0, 0)),
            pl.BlockSpec((1, 32), lambda i: (0, 0)),
            pl.BlockSpec((1, 32), lambda i: (0, 0)),
            pl.BlockSpec((1, 32), lambda i: (0, 0)),
            pl.BlockSpec((1, 32), lambda i: (0, 0)),
            pl.BlockSpec((1, 1), lambda i: (0, 0)),
        ],
        out_specs=pl.BlockSpec((bn, 1), lambda i: (i, 0)),
        out_shape=jax.ShapeDtypeStruct((_B, 1), jnp.float32),
    )


_classifier = _make_classifier()


def kernel(init_feat, basis1, comp1, root1, bias1, basis2, comp2, root2, bias2,
           w1, b1, ln1_g, ln1_b, w2, b2, ln2_g, ln2_b, w3, b3,
           node_indices, edge_index, edge_type):
    f32 = jnp.float32
    i32 = jnp.int32
    src = edge_index[0].astype(i32)
    dst = edge_index[1].astype(i32)
    typ = edge_type.astype(i32)

    pad = _EPAD - _E
    srcp = jnp.concatenate([src, jnp.zeros((pad,), i32)])
    dstp = jnp.concatenate([dst, jnp.full((pad,), _NPAD - 1, i32)])
    typp = jnp.concatenate([typ, jnp.zeros((pad,), i32)])
    dst2d = dstp.reshape(_EPAD // _CH, _CH)
    typ2d = typp.reshape(_EPAD // _CH, _CH)

    table1 = init_feat.reshape(2 * _N, 64)

    aggf1, cntf = _make_agg(True)(table1, srcp, dst2d, typ2d)
    agg1 = aggf1.reshape(2, 2, _NPAD, 64)
    cnt = cntf.reshape(2, 2, _NPAD, 16)

    h2 = _combine1(init_feat, agg1, cnt, root1, basis1, comp1,
                   bias1.reshape(1, -1))

    aggf2, = _make_agg(False)(h2.reshape(2 * _N, 64), srcp, dst2d, typ2d)
    agg2 = aggf2.reshape(2, 2, _NPAD, 64)

    out2 = _combine2(h2, agg2, cnt, root2, basis2, comp2,
                     bias2.reshape(1, -1))

    expert = _make_expert_gather()(out2, node_indices.astype(i32))

    bot = _classifier(expert, w1, b1.reshape(1, -1), ln1_g.reshape(1, -1),
                      ln1_b.reshape(1, -1), w2, b2.reshape(1, -1),
                      ln2_g.reshape(1, -1), ln2_b.reshape(1, -1),
                      w3[:, 0].reshape(1, -1), b3.reshape(1, 1))
    return (expert, bot)


# speedup vs baseline: 1.2774x; 1.2774x over previous
import functools

import jax
import jax.numpy as jnp
from jax import lax
from jax.experimental import pallas as pl
from jax.experimental.pallas import tpu as pltpu
from jax.experimental.pallas import tpu_sc as plsc

_N = 10000
_NPAD = 10240
_ACC = 2 * _NPAD
_E = 320000
_EPAD = 327680
_CH = 128
_EPT = _EPAD // 16
_NCHUNK = _EPT // _CH
_BLKCH = 8
_NBLK = _NCHUNK // _BLKCH
_SLICE = _ACC // 16
_B = 4096

@functools.cache
def _sc_mesh():
    return plsc.VectorSubcoreMesh(core_axis_name="c", subcore_axis_name="s")


@functools.cache
def _make_agg(do_counts):
    out_type = [jax.ShapeDtypeStruct((2 * _ACC, 64), jnp.float32)]
    if do_counts:
        out_type.append(jax.ShapeDtypeStruct((2 * _ACC, 16), jnp.float32))

    blk_e = _BLKCH * _CH
    blk_r = blk_e // _CH
    nring = 2 if do_counts else 4
    scratch = [
        pltpu.VMEM((blk_e,), jnp.int32),
        pltpu.VMEM((blk_r, _CH), jnp.int32),
        pltpu.VMEM((blk_r, _CH), jnp.int32),
        pltpu.VMEM((nring, _CH, 64), jnp.float32),
        pltpu.VMEM_SHARED((_ACC, 64), jnp.float32),
        pltpu.SemaphoreType.DMA,
        pltpu.SemaphoreType.DMA,
    ]
    if do_counts:
        scratch += [
            pltpu.VMEM((_CH, 16), jnp.float32),
            pltpu.VMEM((_CH, 16), jnp.float32),
            pltpu.VMEM_SHARED((_ACC, 16), jnp.float32),
            pltpu.SemaphoreType.DMA,
        ]

    def body(table, srch, dsth, typh, *rest):
        if do_counts:
            (out_agg, out_cnt, srcb, dstb, typb, rows, acc, sem_g,
             sem_s, ones, z16, cntacc, sem_c) = rest
        else:
            out_agg, srcb, dstb, typb, rows, acc, sem_g, sem_s = rest

        c = lax.axis_index("c")
        s = lax.axis_index("s")

        zeros16 = jnp.zeros((16,), jnp.float32)
        ones16 = jnp.ones((16,), jnp.float32)

        def init_buf(i, carry):
            for j in range(4):
                rows[0, i, pl.ds(16 * j, 16)] = zeros16
            if do_counts:
                ones[i, pl.ds(0, 16)] = ones16
                z16[i, pl.ds(0, 16)] = zeros16
            return carry

        lax.fori_loop(0, _CH, init_buf, 0)

        base_e = s * _EPT
        base_r = s * (_EPT // _CH)

        def zero_acc(i, carry):
            base = s * _SLICE + i * _CH
            pltpu.sync_copy(rows.at[0], acc.at[pl.ds(base, _CH)])
            if do_counts:
                pltpu.sync_copy(z16, cntacc.at[pl.ds(base, _CH)])
            return carry

        lax.fori_loop(0, _SLICE // _CH, zero_acc, 0)
        plsc.subcore_barrier()

        halfblk = _NBLK // 2

        def block(blk, carry):
            e0 = base_e + blk * blk_e
            r0 = base_r + blk * blk_r
            pltpu.sync_copy(srch.at[pl.ds(e0, blk_e)], srcb)
            pltpu.sync_copy(dsth.at[pl.ds(r0, blk_r)], dstb)
            pltpu.sync_copy(typh.at[pl.ds(r0, blk_r)], typb)
            for v in range(blk_e // 16):
                sl = pl.ds(16 * v, 16)
                slm = pl.ds(16 * (v % 8), 16)
                srcb[sl] = srcb[sl] + c * _N
                dstb[v // 8, slm] = typb[v // 8, slm] * _NPAD + dstb[v // 8, slm]
            combb = dstb
            if do_counts:
                counting = (((c == 0) & (blk < halfblk))
                            | ((c == 1) & (blk >= halfblk)))

                @pl.when(counting)
                def _cnt_fire():
                    for j in range(_BLKCH):
                        pltpu.async_copy(ones, cntacc.at[combb.at[j]],
                                         sem_c, add=True)
            gd = [None] * _BLKCH
            sd = [None] * _BLKCH
            for j in range(_BLKCH):
                if j >= nring:
                    sd[j - nring].wait()
                gd[j] = pltpu.async_copy(
                    table.at[srcb.at[pl.ds(j * _CH, _CH)]],
                    rows.at[j % nring], sem_g)
                if j >= 1:
                    gd[j - 1].wait()
                    sd[j - 1] = pltpu.async_copy(
                        rows.at[(j - 1) % nring], acc.at[combb.at[j - 1]],
                        sem_s, add=True)
            gd[_BLKCH - 1].wait()
            sd[_BLKCH - 1] = pltpu.async_copy(
                rows.at[(_BLKCH - 1) % nring], acc.at[combb.at[_BLKCH - 1]],
                sem_s, add=True)
            for j in range(_BLKCH - nring, _BLKCH):
                sd[j].wait()
            if do_counts:
                @pl.when(counting)
                def _cnt_drain():
                    for j in range(_BLKCH):
                        pltpu.make_async_copy(
                            ones, cntacc.at[combb.at[j]], sem_c).wait()
            return carry

        lax.fori_loop(0, _NBLK, block, 0)
        plsc.subcore_barrier()

        obase = c * _ACC + s * _SLICE
        pltpu.sync_copy(acc.at[pl.ds(s * _SLICE, _SLICE)],
                        out_agg.at[pl.ds(obase, _SLICE)])
        if do_counts:
            pltpu.sync_copy(cntacc.at[pl.ds(s * _SLICE, _SLICE)],
                            out_cnt.at[pl.ds(obase, _SLICE)])

    return pl.kernel(body, out_type, mesh=_sc_mesh(), scratch_types=scratch,
                     compiler_params=pltpu.CompilerParams(
                         use_tc_tiling_on_sc=False))


@functools.cache
def _make_expert_gather():
    rows_per_tile = _B // 32

    def body(table, idxh, out, idxb, rows, sem):
        c = lax.axis_index("c")
        s = lax.axis_index("s")
        wid = s * 2 + c
        base = wid * rows_per_tile
        pltpu.sync_copy(idxh.at[pl.ds(base, rows_per_tile)], idxb)
        pltpu.async_copy(table.at[idxb], rows, sem).wait()
        pltpu.sync_copy(rows, out.at[pl.ds(base, rows_per_tile)])

    return pl.kernel(
        body,
        jax.ShapeDtypeStruct((_B, 64), jnp.float32),
        mesh=_sc_mesh(),
        scratch_types=[
            pltpu.VMEM((rows_per_tile,), jnp.int32),
            pltpu.VMEM((rows_per_tile, 64), jnp.float32),
            pltpu.SemaphoreType.DMA,
        ],
        compiler_params=pltpu.CompilerParams(use_tc_tiling_on_sc=False),
    )


def _combine_body(x_ref, agg_ref, cnt_ref, root_ref, basis_ref, comp_ref,
                  b_ref, o_ref, *, relu, split_out, x_split):
    if x_split:
        x = jnp.concatenate([x_ref[0], x_ref[1]], axis=1)
    else:
        x = x_ref[...]
    acc = jnp.dot(x, root_ref[...], preferred_element_type=jnp.float32)
    acc = acc + b_ref[...]
    for r in range(2):
        w_r = (comp_ref[r:r + 1, 0:1] * basis_ref[0]
               + comp_ref[r:r + 1, 1:2] * basis_ref[1])
        cr = cnt_ref[0, r, :, 0:1] + cnt_ref[1, r, :, 0:1]
        denom = jnp.maximum(cr, 1.0)
        for h in range(2):
            mean = agg_ref[h, r] / denom
            acc = acc + jnp.dot(mean, w_r[64 * h:64 * (h + 1), :],
                                preferred_element_type=jnp.float32)
    if relu:
        acc = jnp.maximum(acc, 0.0)
    if split_out:
        o_ref[0] = acc[:, :64]
        o_ref[1] = acc[:, 64:]
    else:
        o_ref[...] = acc


def _make_combine(d_out, relu, split_out, x_split, bn=1000):
    grid = (_N // bn,)
    if split_out:
        out_shape = jax.ShapeDtypeStruct((2, _N, 64), jnp.float32)
        out_spec = pl.BlockSpec((2, bn, 64), lambda i: (0, i, 0))
    else:
        out_shape = jax.ShapeDtypeStruct((_N, d_out), jnp.float32)
        out_spec = pl.BlockSpec((bn, d_out), lambda i: (i, 0))
    if x_split:
        x_spec = pl.BlockSpec((2, bn, 64), lambda i: (0, i, 0))
    else:
        x_spec = pl.BlockSpec((bn, 128), lambda i: (i, 0))
    return pl.pallas_call(
        functools.partial(_combine_body, relu=relu, split_out=split_out,
                          x_split=x_split),
        grid=grid,
        in_specs=[
            x_spec,
            pl.BlockSpec((2, 2, bn, 64), lambda i: (0, 0, i, 0)),
            pl.BlockSpec((2, 2, bn, 16), lambda i: (0, 0, i, 0)),
            pl.BlockSpec((128, d_out), lambda i: (0, 0)),
            pl.BlockSpec((2, 128, d_out), lambda i: (0, 0, 0)),
            pl.BlockSpec((2, 2), lambda i: (0, 0)),
            pl.BlockSpec((1, d_out), lambda i: (0, 0)),
        ],
        out_specs=out_spec,
        out_shape=out_shape,
    )


_combine1 = _make_combine(128, relu=True, split_out=True, x_split=False)
_combine2 = _make_combine(64, relu=False, split_out=False, x_split=True)


def _classifier_body(e_ref, w1_ref, b1_ref, g1_ref, bb1_ref,
                     w2_ref, b2_ref, g2_ref, bb2_ref, w3_ref, b3_ref, o_ref):
    z = jnp.dot(e_ref[...], w1_ref[...], preferred_element_type=jnp.float32)
    z = z + b1_ref[...]
    mu = jnp.mean(z, axis=1, keepdims=True)
    zc = z - mu
    var = jnp.mean(zc * zc, axis=1, keepdims=True)
    z = zc * lax.rsqrt(var + 1e-5) * g1_ref[...] + bb1_ref[...]
    z = jnp.maximum(z, 0.0)
    z = jnp.dot(z, w2_ref[...], preferred_element_type=jnp.float32)
    z = z + b2_ref[...]
    mu = jnp.mean(z, axis=1, keepdims=True)
    zc = z - mu
    var = jnp.mean(zc * zc, axis=1, keepdims=True)
    z = zc * lax.rsqrt(var + 1e-5) * g2_ref[...] + bb2_ref[...]
    z = jnp.maximum(z, 0.0)
    p = jnp.sum(z * w3_ref[...], axis=1, keepdims=True) + b3_ref[...]
    o_ref[...] = jax.nn.sigmoid(p)


def _make_classifier(bn=512):
    grid = (_B // bn,)
    return pl.pallas_call(
        _classifier_body,
        grid=grid,
        in_specs=[
            pl.BlockSpec((bn, 64), lambda i: (i, 0)),
            pl.BlockSpec((64, 64), lambda i: (0, 0)),
            pl.BlockSpec((1, 64), lambda i: (0, 0)),
            pl.BlockSpec((1, 64), lambda i: (0, 0)),
            pl.BlockSpec((1, 64), lambda i: (0, 0)),
            pl.BlockSpec((64, 32), lambda i: (0, 0)),
            pl.BlockSpec((1, 32), lambda i: (0, 0)),
            pl.BlockSpec((1, 32), lambda i: (0, 0)),
            pl.BlockSpec((1, 32), lambda i: (0, 0)),
            pl.BlockSpec((1, 32), lambda i: (0, 0)),
            pl.BlockSpec((1, 1), lambda i: (0, 0)),
        ],
        out_specs=pl.BlockSpec((bn, 1), lambda i: (i, 0)),
        out_shape=jax.ShapeDtypeStruct((_B, 1), jnp.float32),
    )


_classifier = _make_classifier()


def kernel(init_feat, basis1, comp1, root1, bias1, basis2, comp2, root2, bias2,
           w1, b1, ln1_g, ln1_b, w2, b2, ln2_g, ln2_b, w3, b3,
           node_indices, edge_index, edge_type):
    f32 = jnp.float32
    i32 = jnp.int32
    src = edge_index[0].astype(i32)
    dst = edge_index[1].astype(i32)
    typ = edge_type.astype(i32)

    pad = _EPAD - _E
    srcp = jnp.concatenate([src, jnp.zeros((pad,), i32)])
    dstp = jnp.concatenate([dst, jnp.full((pad,), _NPAD - 1, i32)])
    typp = jnp.concatenate([typ, jnp.zeros((pad,), i32)])
    dst2d = dstp.reshape(_EPAD // _CH, _CH)
    typ2d = typp.reshape(_EPAD // _CH, _CH)

    table1 = init_feat.reshape(_N, 2, 64).transpose(1, 0, 2).reshape(2 * _N, 64)

    aggf1, cntf = _make_agg(True)(table1, srcp, dst2d, typ2d)
    agg1 = aggf1.reshape(2, 2, _NPAD, 64)
    cnt = cntf.reshape(2, 2, _NPAD, 16)

    h2 = _combine1(init_feat, agg1, cnt, root1, basis1, comp1,
                   bias1.reshape(1, -1))

    aggf2, = _make_agg(False)(h2.reshape(2 * _N, 64), srcp, dst2d, typ2d)
    agg2 = aggf2.reshape(2, 2, _NPAD, 64)

    out2 = _combine2(h2, agg2, cnt, root2, basis2, comp2,
                     bias2.reshape(1, -1))

    expert = _make_expert_gather()(out2, node_indices.astype(i32))

    bot = _classifier(expert, w1, b1.reshape(1, -1), ln1_g.reshape(1, -1),
                      ln1_b.reshape(1, -1), w2, b2.reshape(1, -1),
                      ln2_g.reshape(1, -1), ln2_b.reshape(1, -1),
                      w3[:, 0].reshape(1, -1), b3.reshape(1, 1))
    return (expert, bot)


# speedup vs baseline: 1.2925x; 1.0118x over previous
import functools

import jax
import jax.numpy as jnp
from jax import lax
from jax.experimental import pallas as pl
from jax.experimental.pallas import tpu as pltpu
from jax.experimental.pallas import tpu_sc as plsc

_N = 10000
_NPAD = 10240
_ACC = 2 * _NPAD
_E = 320000
_EPAD = 327680
_CH = 128
_EPT = _EPAD // 16
_NCHUNK = _EPT // _CH
_BLKCH = 8
_NBLK = _NCHUNK // _BLKCH
_SLICE = _ACC // 16
_B = 4096

@functools.cache
def _sc_mesh():
    return plsc.VectorSubcoreMesh(core_axis_name="c", subcore_axis_name="s")


@functools.cache
def _make_agg(do_counts):
    out_type = [jax.ShapeDtypeStruct((2 * _ACC, 64), jnp.float32)]
    if do_counts:
        out_type.append(jax.ShapeDtypeStruct((2 * _ACC, 16), jnp.float32))

    blk_e = _BLKCH * _CH
    blk_r = blk_e // _CH
    nring = 2 if do_counts else 4
    scratch = [
        pltpu.VMEM((blk_e,), jnp.int32),
        pltpu.VMEM((blk_r, _CH), jnp.int32),
        pltpu.VMEM((blk_r, _CH), jnp.int32),
        pltpu.VMEM((nring, _CH, 64), jnp.float32),
        pltpu.VMEM_SHARED((_ACC, 64), jnp.float32),
        pltpu.SemaphoreType.DMA,
        pltpu.SemaphoreType.DMA,
    ]
    if do_counts:
        scratch += [
            pltpu.VMEM((_CH, 16), jnp.float32),
            pltpu.VMEM((_CH, 16), jnp.float32),
            pltpu.VMEM_SHARED((_ACC, 16), jnp.float32),
            pltpu.SemaphoreType.DMA,
        ]

    def body(table, srch, dsth, typh, *rest):
        if do_counts:
            (out_agg, out_cnt, srcb, dstb, typb, rows, acc, sem_g,
             sem_s, ones, z16, cntacc, sem_c) = rest
        else:
            out_agg, srcb, dstb, typb, rows, acc, sem_g, sem_s = rest

        c = lax.axis_index("c")
        s = lax.axis_index("s")

        zeros16 = jnp.zeros((16,), jnp.float32)
        ones16 = jnp.ones((16,), jnp.float32)

        def init_buf(i, carry):
            for j in range(4):
                rows[0, i, pl.ds(16 * j, 16)] = zeros16
            if do_counts:
                ones[i, pl.ds(0, 16)] = ones16
                z16[i, pl.ds(0, 16)] = zeros16
            return carry

        lax.fori_loop(0, _CH, init_buf, 0)

        base_e = s * _EPT
        base_r = s * (_EPT // _CH)

        def zero_acc(i, carry):
            base = s * _SLICE + i * _CH
            pltpu.sync_copy(rows.at[0], acc.at[pl.ds(base, _CH)])
            if do_counts:
                pltpu.sync_copy(z16, cntacc.at[pl.ds(base, _CH)])
            return carry

        lax.fori_loop(0, _SLICE // _CH, zero_acc, 0)
        plsc.subcore_barrier()

        halfblk = _NBLK // 2

        def block(blk, carry):
            e0 = base_e + blk * blk_e
            r0 = base_r + blk * blk_r
            pltpu.sync_copy(srch.at[pl.ds(e0, blk_e)], srcb)
            pltpu.sync_copy(dsth.at[pl.ds(r0, blk_r)], dstb)
            pltpu.sync_copy(typh.at[pl.ds(r0, blk_r)], typb)
            for v in range(blk_e // 16):
                sl = pl.ds(16 * v, 16)
                slm = pl.ds(16 * (v % 8), 16)
                srcb[sl] = srcb[sl] + c * _N
                dstb[v // 8, slm] = typb[v // 8, slm] * _NPAD + dstb[v // 8, slm]
            combb = dstb
            if do_counts:
                counting = (((c == 0) & (blk < halfblk))
                            | ((c == 1) & (blk >= halfblk)))

                @pl.when(counting)
                def _cnt_fire():
                    for j in range(_BLKCH):
                        pltpu.async_copy(ones, cntacc.at[combb.at[j]],
                                         sem_c, add=True)
            gd = [None] * _BLKCH
            sd = [None] * _BLKCH
            lead = nring // 2
            for t in range(_BLKCH + lead):
                if t < _BLKCH:
                    if t >= nring:
                        sd[t - nring].wait()
                    gd[t] = pltpu.async_copy(
                        table.at[srcb.at[pl.ds(t * _CH, _CH)]],
                        rows.at[t % nring], sem_g)
                jj = t - lead
                if 0 <= jj < _BLKCH:
                    gd[jj].wait()
                    sd[jj] = pltpu.async_copy(
                        rows.at[jj % nring], acc.at[combb.at[jj]],
                        sem_s, add=True)
            for j in range(_BLKCH - nring, _BLKCH):
                sd[j].wait()
            if do_counts:
                @pl.when(counting)
                def _cnt_drain():
                    for j in range(_BLKCH):
                        pltpu.make_async_copy(
                            ones, cntacc.at[combb.at[j]], sem_c).wait()
            return carry

        lax.fori_loop(0, _NBLK, block, 0)
        plsc.subcore_barrier()

        obase = c * _ACC + s * _SLICE
        pltpu.sync_copy(acc.at[pl.ds(s * _SLICE, _SLICE)],
                        out_agg.at[pl.ds(obase, _SLICE)])
        if do_counts:
            pltpu.sync_copy(cntacc.at[pl.ds(s * _SLICE, _SLICE)],
                            out_cnt.at[pl.ds(obase, _SLICE)])

    return pl.kernel(body, out_type, mesh=_sc_mesh(), scratch_types=scratch,
                     compiler_params=pltpu.CompilerParams(
                         use_tc_tiling_on_sc=False))


@functools.cache
def _make_expert_gather():
    rows_per_tile = _B // 32

    def body(table, idxh, out, idxb, rows, sem):
        c = lax.axis_index("c")
        s = lax.axis_index("s")
        wid = s * 2 + c
        base = wid * rows_per_tile
        pltpu.sync_copy(idxh.at[pl.ds(base, rows_per_tile)], idxb)
        pltpu.async_copy(table.at[idxb], rows, sem).wait()
        pltpu.sync_copy(rows, out.at[pl.ds(base, rows_per_tile)])

    return pl.kernel(
        body,
        jax.ShapeDtypeStruct((_B, 64), jnp.float32),
        mesh=_sc_mesh(),
        scratch_types=[
            pltpu.VMEM((rows_per_tile,), jnp.int32),
            pltpu.VMEM((rows_per_tile, 64), jnp.float32),
            pltpu.SemaphoreType.DMA,
        ],
        compiler_params=pltpu.CompilerParams(use_tc_tiling_on_sc=False),
    )


def _combine_body(x_ref, agg_ref, cnt_ref, root_ref, basis_ref, comp_ref,
                  b_ref, o_ref, *, relu, split_out, x_split):
    if x_split:
        x = jnp.concatenate([x_ref[0], x_ref[1]], axis=1)
    else:
        x = x_ref[...]
    acc = jnp.dot(x, root_ref[...], preferred_element_type=jnp.float32)
    acc = acc + b_ref[...]
    for r in range(2):
        w_r = (comp_ref[r:r + 1, 0:1] * basis_ref[0]
               + comp_ref[r:r + 1, 1:2] * basis_ref[1])
        cr = cnt_ref[0, r, :, 0:1] + cnt_ref[1, r, :, 0:1]
        denom = jnp.maximum(cr, 1.0)
        for h in range(2):
            mean = agg_ref[h, r] / denom
            acc = acc + jnp.dot(mean, w_r[64 * h:64 * (h + 1), :],
                                preferred_element_type=jnp.float32)
    if relu:
        acc = jnp.maximum(acc, 0.0)
    if split_out:
        o_ref[0] = acc[:, :64]
        o_ref[1] = acc[:, 64:]
    else:
        o_ref[...] = acc


def _make_combine(d_out, relu, split_out, x_split, bn=1000):
    grid = (_N // bn,)
    if split_out:
        out_shape = jax.ShapeDtypeStruct((2, _N, 64), jnp.float32)
        out_spec = pl.BlockSpec((2, bn, 64), lambda i: (0, i, 0))
    else:
        out_shape = jax.ShapeDtypeStruct((_N, d_out), jnp.float32)
        out_spec = pl.BlockSpec((bn, d_out), lambda i: (i, 0))
    if x_split:
        x_spec = pl.BlockSpec((2, bn, 64), lambda i: (0, i, 0))
    else:
        x_spec = pl.BlockSpec((bn, 128), lambda i: (i, 0))
    return pl.pallas_call(
        functools.partial(_combine_body, relu=relu, split_out=split_out,
                          x_split=x_split),
        grid=grid,
        in_specs=[
            x_spec,
            pl.BlockSpec((2, 2, bn, 64), lambda i: (0, 0, i, 0)),
            pl.BlockSpec((2, 2, bn, 16), lambda i: (0, 0, i, 0)),
            pl.BlockSpec((128, d_out), lambda i: (0, 0)),
            pl.BlockSpec((2, 128, d_out), lambda i: (0, 0, 0)),
            pl.BlockSpec((2, 2), lambda i: (0, 0)),
            pl.BlockSpec((1, d_out), lambda i: (0, 0)),
        ],
        out_specs=out_spec,
        out_shape=out_shape,
    )


_combine1 = _make_combine(128, relu=True, split_out=True, x_split=False)
_combine2 = _make_combine(64, relu=False, split_out=False, x_split=True)


def _classifier_body(e_ref, w1_ref, b1_ref, g1_ref, bb1_ref,
                     w2_ref, b2_ref, g2_ref, bb2_ref, w3_ref, b3_ref, o_ref):
    z = jnp.dot(e_ref[...], w1_ref[...], preferred_element_type=jnp.float32)
    z = z + b1_ref[...]
    mu = jnp.mean(z, axis=1, keepdims=True)
    zc = z - mu
    var = jnp.mean(zc * zc, axis=1, keepdims=True)
    z = zc * lax.rsqrt(var + 1e-5) * g1_ref[...] + bb1_ref[...]
    z = jnp.maximum(z, 0.0)
    z = jnp.dot(z, w2_ref[...], preferred_element_type=jnp.float32)
    z = z + b2_ref[...]
    mu = jnp.mean(z, axis=1, keepdims=True)
    zc = z - mu
    var = jnp.mean(zc * zc, axis=1, keepdims=True)
    z = zc * lax.rsqrt(var + 1e-5) * g2_ref[...] + bb2_ref[...]
    z = jnp.maximum(z, 0.0)
    p = jnp.sum(z * w3_ref[...], axis=1, keepdims=True) + b3_ref[...]
    o_ref[...] = jax.nn.sigmoid(p)


def _make_classifier(bn=512):
    grid = (_B // bn,)
    return pl.pallas_call(
        _classifier_body,
        grid=grid,
        in_specs=[
            pl.BlockSpec((bn, 64), lambda i: (i, 0)),
            pl.BlockSpec((64, 64), lambda i: (0, 0)),
            pl.BlockSpec((1, 64), lambda i: (0, 0)),
            pl.BlockSpec((1, 64), lambda i: (0, 0)),
            pl.BlockSpec((1, 64), lambda i: (0, 0)),
            pl.BlockSpec((64, 32), lambda i: (0, 0)),
            pl.BlockSpec((1, 32), lambda i: (0, 0)),
            pl.BlockSpec((1, 32), lambda i: (0, 0)),
            pl.BlockSpec((1, 32), lambda i: (0, 0)),
            pl.BlockSpec((1, 32), lambda i: (0, 0)),
            pl.BlockSpec((1, 1), lambda i: (0, 0)),
        ],
        out_specs=pl.BlockSpec((bn, 1), lambda i: (i, 0)),
        out_shape=jax.ShapeDtypeStruct((_B, 1), jnp.float32),
    )


_classifier = _make_classifier()


def kernel(init_feat, basis1, comp1, root1, bias1, basis2, comp2, root2, bias2,
           w1, b1, ln1_g, ln1_b, w2, b2, ln2_g, ln2_b, w3, b3,
           node_indices, edge_index, edge_type):
    f32 = jnp.float32
    i32 = jnp.int32
    src = edge_index[0].astype(i32)
    dst = edge_index[1].astype(i32)
    typ = edge_type.astype(i32)

    pad = _EPAD - _E
    srcp = jnp.concatenate([src, jnp.zeros((pad,), i32)])
    dstp = jnp.concatenate([dst, jnp.full((pad,), _NPAD - 1, i32)])
    typp = jnp.concatenate([typ, jnp.zeros((pad,), i32)])
    dst2d = dstp.reshape(_EPAD // _CH, _CH)
    typ2d = typp.reshape(_EPAD // _CH, _CH)

    table1 = init_feat.reshape(_N, 2, 64).transpose(1, 0, 2).reshape(2 * _N, 64)

    aggf1, cntf = _make_agg(True)(table1, srcp, dst2d, typ2d)
    agg1 = aggf1.reshape(2, 2, _NPAD, 64)
    cnt = cntf.reshape(2, 2, _NPAD, 16)

    h2 = _combine1(init_feat, agg1, cnt, root1, basis1, comp1,
                   bias1.reshape(1, -1))

    aggf2, = _make_agg(False)(h2.reshape(2 * _N, 64), srcp, dst2d, typ2d)
    agg2 = aggf2.reshape(2, 2, _NPAD, 64)

    out2 = _combine2(h2, agg2, cnt, root2, basis2, comp2,
                     bias2.reshape(1, -1))

    expert = _make_expert_gather()(out2, node_indices.astype(i32))

    bot = _classifier(expert, w1, b1.reshape(1, -1), ln1_g.reshape(1, -1),
                      ln1_b.reshape(1, -1), w2, b2.reshape(1, -1),
                      ln2_g.reshape(1, -1), ln2_b.reshape(1, -1),
                      w3[:, 0].reshape(1, -1), b3.reshape(1, 1))
    return (expert, bot)


# speedup vs baseline: 1.2938x; 1.0011x over previous
import functools

import jax
import jax.numpy as jnp
from jax import lax
from jax.experimental import pallas as pl
from jax.experimental.pallas import tpu as pltpu
from jax.experimental.pallas import tpu_sc as plsc

_N = 10000
_NPAD = 10240
_ACC = 2 * _NPAD
_E = 320000
_EPAD = 327680
_CH = 128
_EPT = _EPAD // 16
_NCHUNK = _EPT // _CH
_BLKCH = 8
_NBLK = _NCHUNK // _BLKCH
_SLICE = _ACC // 16
_B = 4096

@functools.cache
def _sc_mesh():
    return plsc.VectorSubcoreMesh(core_axis_name="c", subcore_axis_name="s")


@functools.cache
def _make_agg(do_counts):
    out_type = [jax.ShapeDtypeStruct((2 * _ACC, 64), jnp.float32)]
    if do_counts:
        out_type.append(jax.ShapeDtypeStruct((2 * _ACC, 16), jnp.float32))

    blk_e = _BLKCH * _CH
    blk_r = blk_e // _CH
    nring = 2 if do_counts else 4
    scratch = [
        pltpu.VMEM((blk_e,), jnp.int32),
        pltpu.VMEM((blk_r, _CH), jnp.int32),
        pltpu.VMEM((blk_r, _CH), jnp.int32),
        pltpu.VMEM((nring, _CH, 64), jnp.float32),
        pltpu.VMEM_SHARED((_ACC, 64), jnp.float32),
        pltpu.SemaphoreType.DMA,
        pltpu.SemaphoreType.DMA,
    ]
    if do_counts:
        scratch += [
            pltpu.VMEM((_CH, 16), jnp.float32),
            pltpu.VMEM((_CH, 16), jnp.float32),
            pltpu.VMEM_SHARED((_ACC, 16), jnp.float32),
            pltpu.SemaphoreType.DMA,
        ]

    def body(table, srch, dsth, typh, *rest):
        if do_counts:
            (out_agg, out_cnt, srcb, dstb, typb, rows, acc, sem_g,
             sem_s, ones, z16, cntacc, sem_c) = rest
        else:
            out_agg, srcb, dstb, typb, rows, acc, sem_g, sem_s = rest

        c = lax.axis_index("c")
        s = lax.axis_index("s")

        zeros16 = jnp.zeros((16,), jnp.float32)
        ones16 = jnp.ones((16,), jnp.float32)

        def init_buf(i, carry):
            for j in range(4):
                rows[0, i, pl.ds(16 * j, 16)] = zeros16
            if do_counts:
                ones[i, pl.ds(0, 16)] = ones16
                z16[i, pl.ds(0, 16)] = zeros16
            return carry

        lax.fori_loop(0, _CH, init_buf, 0)

        base_e = s * _EPT
        base_r = s * (_EPT // _CH)

        def zero_acc(i, carry):
            base = s * _SLICE + i * _CH
            pltpu.sync_copy(rows.at[0], acc.at[pl.ds(base, _CH)])
            if do_counts:
                pltpu.sync_copy(z16, cntacc.at[pl.ds(base, _CH)])
            return carry

        lax.fori_loop(0, _SLICE // _CH, zero_acc, 0)
        plsc.subcore_barrier()

        halfblk = _NBLK // 2

        def block(blk, carry):
            e0 = base_e + blk * blk_e
            r0 = base_r + blk * blk_r
            pltpu.sync_copy(srch.at[pl.ds(e0, blk_e)], srcb)
            pltpu.sync_copy(dsth.at[pl.ds(r0, blk_r)], dstb)
            pltpu.sync_copy(typh.at[pl.ds(r0, blk_r)], typb)
            for v in range(blk_e // 16):
                sl = pl.ds(16 * v, 16)
                slm = pl.ds(16 * (v % 8), 16)
                srcb[sl] = srcb[sl] + c * _N
                dstb[v // 8, slm] = typb[v // 8, slm] * _NPAD + dstb[v // 8, slm]
            combb = dstb
            if do_counts:
                counting = (((c == 0) & (blk < halfblk))
                            | ((c == 1) & (blk >= halfblk)))

                @pl.when(counting)
                def _cnt_fire():
                    for j in range(_BLKCH):
                        pltpu.async_copy(ones, cntacc.at[combb.at[j]],
                                         sem_c, add=True)
            gd = [None] * _BLKCH
            sd = [None] * _BLKCH
            lead = nring // 2
            for t in range(_BLKCH + lead):
                if t < _BLKCH:
                    if t >= nring:
                        sd[t - nring].wait()
                    gd[t] = pltpu.async_copy(
                        table.at[srcb.at[pl.ds(t * _CH, _CH)]],
                        rows.at[t % nring], sem_g)
                jj = t - lead
                if 0 <= jj < _BLKCH:
                    gd[jj].wait()
                    sd[jj] = pltpu.async_copy(
                        rows.at[jj % nring], acc.at[combb.at[jj]],
                        sem_s, add=True)
            for j in range(_BLKCH - nring, _BLKCH):
                sd[j].wait()
            if do_counts:
                @pl.when(counting)
                def _cnt_drain():
                    for j in range(_BLKCH):
                        pltpu.make_async_copy(
                            ones, cntacc.at[combb.at[j]], sem_c).wait()
            return carry

        lax.fori_loop(0, _NBLK, block, 0)
        plsc.subcore_barrier()

        obase = c * _ACC + s * _SLICE
        pltpu.sync_copy(acc.at[pl.ds(s * _SLICE, _SLICE)],
                        out_agg.at[pl.ds(obase, _SLICE)])
        if do_counts:
            pltpu.sync_copy(cntacc.at[pl.ds(s * _SLICE, _SLICE)],
                            out_cnt.at[pl.ds(obase, _SLICE)])

    return pl.kernel(body, out_type, mesh=_sc_mesh(), scratch_types=scratch,
                     compiler_params=pltpu.CompilerParams(
                         use_tc_tiling_on_sc=False))


@functools.cache
def _make_expert_gather():
    rows_per_tile = _B // 32

    def body(table, idxh, out, idxb, rows, sem):
        c = lax.axis_index("c")
        s = lax.axis_index("s")
        wid = s * 2 + c
        base = wid * rows_per_tile
        pltpu.sync_copy(idxh.at[pl.ds(base, rows_per_tile)], idxb)
        pltpu.async_copy(table.at[idxb], rows, sem).wait()
        pltpu.sync_copy(rows, out.at[pl.ds(base, rows_per_tile)])

    return pl.kernel(
        body,
        jax.ShapeDtypeStruct((_B, 64), jnp.float32),
        mesh=_sc_mesh(),
        scratch_types=[
            pltpu.VMEM((rows_per_tile,), jnp.int32),
            pltpu.VMEM((rows_per_tile, 64), jnp.float32),
            pltpu.SemaphoreType.DMA,
        ],
        compiler_params=pltpu.CompilerParams(use_tc_tiling_on_sc=False),
    )


def _combine_body(x_ref, agg_ref, cnt_ref, root_ref, basis_ref, comp_ref,
                  b_ref, o_ref, *, relu, split_out, x_split):
    if x_split:
        x = jnp.concatenate([x_ref[0], x_ref[1]], axis=1)
    else:
        x = x_ref[...]
    acc = jnp.dot(x, root_ref[...], preferred_element_type=jnp.float32)
    acc = acc + b_ref[...]
    for r in range(2):
        w_r = (comp_ref[r:r + 1, 0:1] * basis_ref[0]
               + comp_ref[r:r + 1, 1:2] * basis_ref[1])
        cr = cnt_ref[0, r, :, 0:1] + cnt_ref[1, r, :, 0:1]
        denom = jnp.maximum(cr, 1.0)
        for h in range(2):
            mean = agg_ref[h, r] / denom
            acc = acc + jnp.dot(mean, w_r[64 * h:64 * (h + 1), :],
                                preferred_element_type=jnp.float32)
    if relu:
        acc = jnp.maximum(acc, 0.0)
    if split_out:
        o_ref[0] = acc[:, :64]
        o_ref[1] = acc[:, 64:]
    else:
        o_ref[...] = acc


def _make_combine(d_out, relu, split_out, x_split, bn=1000):
    grid = (_N // bn,)
    if split_out:
        out_shape = jax.ShapeDtypeStruct((2, _N, 64), jnp.float32)
        out_spec = pl.BlockSpec((2, bn, 64), lambda i: (0, i, 0))
    else:
        out_shape = jax.ShapeDtypeStruct((_N, d_out), jnp.float32)
        out_spec = pl.BlockSpec((bn, d_out), lambda i: (i, 0))
    if x_split:
        x_spec = pl.BlockSpec((2, bn, 64), lambda i: (0, i, 0))
    else:
        x_spec = pl.BlockSpec((bn, 128), lambda i: (i, 0))
    return pl.pallas_call(
        functools.partial(_combine_body, relu=relu, split_out=split_out,
                          x_split=x_split),
        grid=grid,
        in_specs=[
            x_spec,
            pl.BlockSpec((2, 2, bn, 64), lambda i: (0, 0, i, 0)),
            pl.BlockSpec((2, 2, bn, 16), lambda i: (0, 0, i, 0)),
            pl.BlockSpec((128, d_out), lambda i: (0, 0)),
            pl.BlockSpec((2, 128, d_out), lambda i: (0, 0, 0)),
            pl.BlockSpec((2, 2), lambda i: (0, 0)),
            pl.BlockSpec((1, d_out), lambda i: (0, 0)),
        ],
        out_specs=out_spec,
        out_shape=out_shape,
    )


_combine1 = _make_combine(128, relu=True, split_out=True, x_split=False)
_combine2 = _make_combine(64, relu=False, split_out=False, x_split=True)


def _classifier_body(e_ref, w1_ref, b1_ref, g1_ref, bb1_ref,
                     w2_ref, b2_ref, g2_ref, bb2_ref, w3_ref, b3_ref, o_ref):
    z = jnp.dot(e_ref[...], w1_ref[...], preferred_element_type=jnp.float32)
    z = z + b1_ref[...]
    mu = jnp.mean(z, axis=1, keepdims=True)
    zc = z - mu
    var = jnp.mean(zc * zc, axis=1, keepdims=True)
    z = zc * lax.rsqrt(var + 1e-5) * g1_ref[...] + bb1_ref[...]
    z = jnp.maximum(z, 0.0)
    z = jnp.dot(z, w2_ref[...], preferred_element_type=jnp.float32)
    z = z + b2_ref[...]
    mu = jnp.mean(z, axis=1, keepdims=True)
    zc = z - mu
    var = jnp.mean(zc * zc, axis=1, keepdims=True)
    z = zc * lax.rsqrt(var + 1e-5) * g2_ref[...] + bb2_ref[...]
    z = jnp.maximum(z, 0.0)
    p = jnp.sum(z * w3_ref[...], axis=1, keepdims=True) + b3_ref[...]
    o_ref[...] = jax.nn.sigmoid(p)


def _make_classifier(bn=512):
    grid = (_B // bn,)
    return pl.pallas_call(
        _classifier_body,
        grid=grid,
        in_specs=[
            pl.BlockSpec((bn, 64), lambda i: (i, 0)),
            pl.BlockSpec((64, 64), lambda i: (0, 0)),
            pl.BlockSpec((1, 64), lambda i: (0, 0)),
            pl.BlockSpec((1, 64), lambda i: (0, 0)),
            pl.BlockSpec((1, 64), lambda i: (0, 0)),
            pl.BlockSpec((64, 32), lambda i: (0, 0)),
            pl.BlockSpec((1, 32), lambda i: (0, 0)),
            pl.BlockSpec((1, 32), lambda i: (0, 0)),
            pl.BlockSpec((1, 32), lambda i: (0, 0)),
            pl.BlockSpec((1, 32), lambda i: (0, 0)),
            pl.BlockSpec((1, 1), lambda i: (0, 0)),
        ],
        out_specs=pl.BlockSpec((bn, 1), lambda i: (i, 0)),
        out_shape=jax.ShapeDtypeStruct((_B, 1), jnp.float32),
    )


_classifier = _make_classifier()


def kernel(init_feat, basis1, comp1, root1, bias1, basis2, comp2, root2, bias2,
           w1, b1, ln1_g, ln1_b, w2, b2, ln2_g, ln2_b, w3, b3,
           node_indices, edge_index, edge_type):
    i32 = jnp.int32
    src = edge_index[0].astype(i32)
    dst = edge_index[1].astype(i32)
    typ = edge_type.astype(i32)

    pad = _EPAD - _E
    srcp = jnp.concatenate([src, jnp.zeros((pad,), i32)])
    dstp = jnp.concatenate([dst, jnp.full((pad,), _NPAD - 1, i32)])
    typp = jnp.concatenate([typ, jnp.zeros((pad,), i32)])
    dst2d = dstp.reshape(_EPAD // _CH, _CH)
    typ2d = typp.reshape(_EPAD // _CH, _CH)

    table1 = init_feat.reshape(_N, 2, 64).transpose(1, 0, 2).reshape(2 * _N, 64)

    aggf1, cntf = _make_agg(True)(table1, srcp, dst2d, typ2d)
    agg1 = aggf1.reshape(2, 2, _NPAD, 64)
    cnt = cntf.reshape(2, 2, _NPAD, 16)

    h2 = _combine1(init_feat, agg1, cnt, root1, basis1, comp1,
                   bias1.reshape(1, -1))

    aggf2, = _make_agg(False)(h2.reshape(2 * _N, 64), srcp, dst2d, typ2d)
    agg2 = aggf2.reshape(2, 2, _NPAD, 64)

    out2 = _combine2(h2, agg2, cnt, root2, basis2, comp2,
                     bias2.reshape(1, -1))

    expert = _make_expert_gather()(out2, node_indices.astype(i32))

    bot = _classifier(expert, w1, b1.reshape(1, -1), ln1_g.reshape(1, -1),
                      ln1_b.reshape(1, -1), w2, b2.reshape(1, -1),
                      ln2_g.reshape(1, -1), ln2_b.reshape(1, -1),
                      w3[:, 0].reshape(1, -1), b3.reshape(1, 1))
    return (expert, bot)
